# Initial kernel scaffold; baseline (speedup 1.0000x reference)
import jax
import jax.numpy as jnp
from jax.experimental import pallas as pl


def kernel(x, edge_index, hyperedge_index, W1, b1, W2, b2):
    raise NotImplementedError("write your pallas kernel here")


# speedup vs baseline: 3.0681x; 3.0681x over previous
import functools

import jax
import jax.numpy as jnp
from jax import lax
from jax.experimental import pallas as pl
from jax.experimental.pallas import tpu as pltpu
from jax.experimental.pallas import tpu_sc as plsc

N = 10000
NNZ = 320000
D = 128
NC, NS = 2, 16
NW = NC * NS
K = 128
NCHUNK = 80
NNZP = NW * NCHUNK * K
NP = 10112
SP = NP // NS
SPT = SP - 4 * K


def _sc_agg(gidx2d, sidx2d, table, zblk):
    @functools.partial(
        pl.kernel,
        out_type=jax.ShapeDtypeStruct((2 * NP, D), jnp.float32),
        mesh=plsc.VectorSubcoreMesh(core_axis_name="c", subcore_axis_name="s"),
        scratch_types=[
            pltpu.VMEM((K,), jnp.int32),
            pltpu.VMEM((K,), jnp.int32),
            pltpu.VMEM((K, D), jnp.float32),
            pltpu.VMEM_SHARED((NP, D), jnp.float32),
            pltpu.SemaphoreType.DMA,
        ],
    )
    def agg(gidx_hbm, sidx_hbm, table_hbm, z_hbm, out_hbm,
            gi_v, si_v, rows_v, acc_sh, sem):
        c = lax.axis_index("c")
        s = lax.axis_index("s")
        wid = c * NS + s
        for t in range(4):
            pltpu.sync_copy(z_hbm, acc_sh.at[pl.ds(s * SP + t * K, K)])
        pltpu.sync_copy(z_hbm.at[pl.ds(0, SPT)],
                        acc_sh.at[pl.ds(s * SP + 4 * K, SPT)])
        plsc.subcore_barrier()

        def body(j, carry):
            pltpu.sync_copy(gidx_hbm.at[wid * NCHUNK + j], gi_v)
            pltpu.sync_copy(sidx_hbm.at[wid * NCHUNK + j], si_v)
            pltpu.async_copy(table_hbm.at[gi_v], rows_v, sem).wait()
            pltpu.sync_copy(rows_v, acc_sh.at[si_v], add=True)
            return carry

        lax.fori_loop(0, NCHUNK, body, 0)
        plsc.subcore_barrier()
        for t in range(4):
            r0 = s * SP + t * K
            pltpu.sync_copy(acc_sh.at[pl.ds(r0, K)], rows_v)
            pltpu.sync_copy(rows_v, out_hbm.at[pl.ds(c * NP + r0, K)])
        r0 = s * SP + 4 * K
        pltpu.sync_copy(acc_sh.at[pl.ds(r0, SPT)], rows_v.at[pl.ds(0, SPT)])
        pltpu.sync_copy(rows_v.at[pl.ds(0, SPT)],
                        out_hbm.at[pl.ds(c * NP + r0, SPT)])

    return agg(gidx2d, sidx2d, table, zblk)


_RB = 1000
_RBI = 632


def _tc_invp(pv):
    def body(p_ref, o_ref):
        o_ref[...] = 1.0 / jnp.maximum(p_ref[0] + p_ref[1], 1.0)

    return pl.pallas_call(
        body,
        grid=(NP // _RBI,),
        in_specs=[pl.BlockSpec((2, _RBI, D), lambda i: (0, i, 0))],
        out_specs=pl.BlockSpec((_RBI, D), lambda i: (i, 0)),
        out_shape=jax.ShapeDtypeStruct((NP, D), jnp.float32),
    )(pv)


def _tc_matmul(x, w):
    def body(x_ref, w_ref, o_ref):
        o_ref[...] = lax.dot_general(
            x_ref[...], w_ref[...], (((1,), (0,)), ((), ())),
            precision=lax.Precision.HIGHEST,
            preferred_element_type=jnp.float32)

    return pl.pallas_call(
        body,
        grid=(N // _RB,),
        in_specs=[pl.BlockSpec((_RB, D), lambda i: (i, 0)),
                  pl.BlockSpec((D, D), lambda i: (0, 0))],
        out_specs=pl.BlockSpec((_RB, D), lambda i: (i, 0)),
        out_shape=jax.ShapeDtypeStruct((N, D), jnp.float32),
    )(x, w)


def _combine(p_ref, inv_ref):
    q = p_ref[0] + p_ref[1]
    return q * inv_ref[:, 0:1]


def _tc_scale(p2, inv2d):
    def body(p_ref, inv_ref, o_ref):
        o_ref[...] = _combine(p_ref, inv_ref)

    return pl.pallas_call(
        body,
        grid=(N // _RB,),
        in_specs=[pl.BlockSpec((2, _RB, D), lambda i: (0, i, 0)),
                  pl.BlockSpec((_RB, D), lambda i: (i, 0))],
        out_specs=pl.BlockSpec((_RB, D), lambda i: (i, 0)),
        out_shape=jax.ShapeDtypeStruct((N, D), jnp.float32),
    )(p2, inv2d)


def _tc_fuse(p2, inv2d, b, w):
    def body(p_ref, inv_ref, b_ref, w_ref, o_ref):
        h = jnp.maximum(_combine(p_ref, inv_ref) + b_ref[...], 0.0)
        o_ref[...] = lax.dot_general(
            h, w_ref[...], (((1,), (0,)), ((), ())),
            precision=lax.Precision.HIGHEST,
            preferred_element_type=jnp.float32)

    return pl.pallas_call(
        body,
        grid=(N // _RB,),
        in_specs=[pl.BlockSpec((2, _RB, D), lambda i: (0, i, 0)),
                  pl.BlockSpec((_RB, D), lambda i: (i, 0)),
                  pl.BlockSpec((1, D), lambda i: (0, 0)),
                  pl.BlockSpec((D, D), lambda i: (0, 0))],
        out_specs=pl.BlockSpec((_RB, D), lambda i: (i, 0)),
        out_shape=jax.ShapeDtypeStruct((N, D), jnp.float32),
    )(p2, inv2d, b, w)


def _tc_final(p2, inv2d, b):
    def body(p_ref, inv_ref, b_ref, o_ref):
        o_ref[...] = _combine(p_ref, inv_ref) + b_ref[...]

    return pl.pallas_call(
        body,
        grid=(N // _RB,),
        in_specs=[pl.BlockSpec((2, _RB, D), lambda i: (0, i, 0)),
                  pl.BlockSpec((_RB, D), lambda i: (i, 0)),
                  pl.BlockSpec((1, D), lambda i: (0, 0))],
        out_specs=pl.BlockSpec((_RB, D), lambda i: (i, 0)),
        out_shape=jax.ShapeDtypeStruct((N, D), jnp.float32),
    )(p2, inv2d, b)


def kernel(x, edge_index, hyperedge_index, W1, b1, W2, b2):
    del edge_index
    node = hyperedge_index[0]
    he = hyperedge_index[1]
    npad = NNZP - NNZ
    pad_g = jnp.zeros((npad,), jnp.int32)
    pad_s = jnp.full((npad,), N, jnp.int32)
    node_g = jnp.concatenate([node, pad_g]).reshape(NW * NCHUNK, K)
    node_s = jnp.concatenate([node, pad_s]).reshape(NW * NCHUNK, K)
    he_g = jnp.concatenate([he, pad_g]).reshape(NW * NCHUNK, K)
    he_s = jnp.concatenate([he, pad_s]).reshape(NW * NCHUNK, K)
    zblk = jnp.zeros((K, D), jnp.float32)
    b1r = b1.reshape(1, D)
    b2r = b2.reshape(1, D)
    gstack = jnp.stack([node_g, he_g])
    sstack = jnp.stack([he_s, node_s])

    table0 = _tc_matmul(x, W1)
    ones_tab = jnp.ones((N, D), jnp.float32)
    invs0 = jnp.zeros((2, NP, D), jnp.float32)

    def step(t, carry):
        table, invs = carry
        p = t % 2
        g2d = lax.dynamic_index_in_dim(gstack, p, 0, keepdims=False)
        s2d = lax.dynamic_index_in_dim(sstack, p, 0, keepdims=False)
        pv = _sc_agg(g2d, s2d, table, zblk).reshape(2, NP, D)
        return lax.switch(
            t,
            (lambda tb, iv, q: (tb, iv.at[0].set(_tc_invp(q))),
             lambda tb, iv, q: (table0, iv.at[1].set(_tc_invp(q))),
             lambda tb, iv, q: (_tc_scale(q, iv[0]), iv),
             lambda tb, iv, q: (_tc_fuse(q, iv[1], b1r, W2), iv),
             lambda tb, iv, q: (_tc_scale(q, iv[0]), iv),
             lambda tb, iv, q: (_tc_final(q, iv[1], b2r), iv)),
            table, invs, pv)

    out, _ = lax.fori_loop(0, 6, step, (ones_tab, invs0))
    return out
